# baseline (speedup 1.0000x reference)
import functools

import jax
import jax.numpy as jnp
from jax import lax
from jax.experimental import pallas as pl
from jax.experimental.pallas import tpu as pltpu
from jax.experimental.pallas import tpu_sc as plsc

N = 50000
E = 800000
G = 512
H = 128
ROWS = 400

LW = 128
EPAD = 819200
NWIN = EPAD // LW
NACC = 51200
RPT = NACC // 16
CH = 40
NPADROW = NACC - N


def _fold_bn(w, b, g, bb, m, v):
    s = g / jnp.sqrt(v + 1e-5)
    W = w * s[None, :]
    B = b * s + bb - m * s
    return W, B


def _zero_zbuf(zbuf, cols):
    def body(r, _):
        for c16 in range(cols // 16):
            zbuf[r, pl.ds(16 * c16, 16)] = jnp.zeros((16,), jnp.float32)
        return 0
    lax.fori_loop(0, LW, body, 0)


def _zero_accum_slice(zbuf, accum, s):
    def body(k, _):
        pltpu.sync_copy(zbuf, accum.at[pl.ds(s * RPT + k * LW, LW)])
        return 0
    lax.fori_loop(0, RPT // LW, body, 0)


def _sc_aggr_layer0(x16, src2d, dst2d, attr1d, w16, b16):
    mesh = plsc.VectorSubcoreMesh(core_axis_name="c", subcore_axis_name="s")
    wpt = NWIN // 32
    nch = wpt // CH

    @functools.partial(
        pl.kernel, mesh=mesh,
        out_type=jax.ShapeDtypeStruct((2, NACC, 16), jnp.float32),
        compiler_params=pltpu.CompilerParams(use_tc_tiling_on_sc=False),
        scratch_types=[
            pltpu.VMEM((CH, LW), jnp.int32),
            pltpu.VMEM((CH, LW), jnp.int32),
            pltpu.VMEM((CH * LW,), jnp.float32),
            pltpu.VMEM((LW, 16), jnp.float32),
            pltpu.VMEM((LW, 16), jnp.float32),
            pltpu.VMEM((16,), jnp.float32),
            pltpu.VMEM((16,), jnp.float32),
            pltpu.VMEM_SHARED((NACC, 16), jnp.float32),
            pltpu.SemaphoreType.DMA,
        ],
    )
    def k(x_hbm, src_hbm, dst_hbm, attr_hbm, w_hbm, b_hbm, out_hbm,
          srcb, dstb, attrb, rows, zbuf, wbuf, bbuf, accum, sem):
        c = lax.axis_index("c")
        s = lax.axis_index("s")
        wid = s * 2 + c
        pltpu.sync_copy(w_hbm, wbuf)
        pltpu.sync_copy(b_hbm, bbuf)
        wv = wbuf[pl.ds(0, 16)]
        bv = bbuf[pl.ds(0, 16)]
        _zero_zbuf(zbuf, 16)
        _zero_accum_slice(zbuf, accum, s)
        plsc.subcore_barrier()

        def chunk_body(jc, _):
            row0 = wid * wpt + jc * CH
            pltpu.sync_copy(src_hbm.at[pl.ds(row0, CH)], srcb)
            pltpu.sync_copy(dst_hbm.at[pl.ds(row0, CH)], dstb)
            pltpu.sync_copy(attr_hbm.at[pl.ds(row0 * LW, CH * LW)], attrb)

            def win_body(jw, _):
                pltpu.async_copy(x_hbm.at[srcb.at[jw]], rows, sem).wait()

                def grp_body(g, _):
                    av = attrb[pl.ds(jw * LW + g * 16, 16)]
                    for i in range(16):
                        e = g * 16 + i
                        a = av[i]
                        v = rows[e, pl.ds(0, 16)]
                        rows[e, pl.ds(0, 16)] = jnp.maximum(v + a * wv + bv,
                                                            0.0)
                    return 0
                lax.fori_loop(0, LW // 16, grp_body, 0)
                pltpu.sync_copy(rows, accum.at[dstb.at[jw]], add=True)
                return 0
            lax.fori_loop(0, CH, win_body, 0)
            return 0
        lax.fori_loop(0, nch, chunk_body, 0)
        plsc.subcore_barrier()
        pltpu.sync_copy(accum.at[pl.ds(s * RPT, RPT)],
                        out_hbm.at[c, pl.ds(s * RPT, RPT)])

    return k(x16, src2d, dst2d, attr1d, w16, b16)


def _sc_aggr_layer1(h_flat, src2d, dst2d, attr1d, w128, b128):
    mesh = plsc.VectorSubcoreMesh(core_axis_name="c", subcore_axis_name="s")
    wpt = NWIN // 16
    nch = wpt // CH

    @functools.partial(
        pl.kernel, mesh=mesh,
        out_type=jax.ShapeDtypeStruct((4, NACC, 32), jnp.bfloat16),
        compiler_params=pltpu.CompilerParams(use_tc_tiling_on_sc=False,
                                             needs_layout_passes=False),
        scratch_types=[
            pltpu.VMEM((CH, LW), jnp.int32),
            pltpu.VMEM((CH, LW), jnp.int32),
            pltpu.VMEM((CH * LW,), jnp.float32),
            pltpu.VMEM((LW, 32), jnp.float32),
            pltpu.VMEM((LW, 32), jnp.bfloat16),
            pltpu.VMEM((LW, 32), jnp.bfloat16),
            pltpu.VMEM((32,), jnp.float32),
            pltpu.VMEM((32,), jnp.float32),
            pltpu.VMEM_SHARED((NACC, 32), jnp.bfloat16),
            pltpu.SemaphoreType.DMA,
        ],
    )
    def k(h_hbm, src_hbm, dst_hbm, attr_hbm, w_hbm, b_hbm, out_hbm,
          srcb, dstb, attrb, rows, rows_bf, zbuf, wbuf, bbuf, accum, sem):
        c = lax.axis_index("c")
        s = lax.axis_index("s")

        def zrow(r, _):
            zbuf[r, pl.ds(0, 32)] = jnp.zeros((32,), jnp.bfloat16)
            return 0
        lax.fori_loop(0, LW, zrow, 0)
        for bi in range(2):
            blk = c * 2 + bi
            pltpu.sync_copy(w_hbm.at[pl.ds(blk * 32, 32)], wbuf)
            pltpu.sync_copy(b_hbm.at[pl.ds(blk * 32, 32)], bbuf)
            wv0 = wbuf[pl.ds(0, 16)]
            wv1 = wbuf[pl.ds(16, 16)]
            bv0 = bbuf[pl.ds(0, 16)]
            bv1 = bbuf[pl.ds(16, 16)]
            _zero_accum_slice(zbuf, accum, s)
            plsc.subcore_barrier()
            boff = blk * N

            def chunk_body(jc, _):
                row0 = s * wpt + jc * CH
                pltpu.sync_copy(src_hbm.at[pl.ds(row0, CH)], srcb)
                pltpu.sync_copy(dst_hbm.at[pl.ds(row0, CH)], dstb)
                pltpu.sync_copy(attr_hbm.at[pl.ds(row0 * LW, CH * LW)], attrb)

                def adj_body(jw, _):
                    for i8 in range(LW // 16):
                        sl = pl.ds(16 * i8, 16)
                        srcb[jw, sl] = srcb[jw, sl] + boff
                    return 0
                lax.fori_loop(0, CH, adj_body, 0)

                def win_body(jw, _):
                    pltpu.async_copy(h_hbm.at[srcb.at[jw]], rows, sem).wait()

                    def grp_body(g, _):
                        av = attrb[pl.ds(jw * LW + g * 16, 16)]
                        for i in range(16):
                            e = g * 16 + i
                            a = av[i]
                            v0 = rows[e, pl.ds(0, 16)]
                            m0 = jnp.maximum(v0 + a * wv0 + bv0, 0.0)
                            v1 = rows[e, pl.ds(16, 16)]
                            m1 = jnp.maximum(v1 + a * wv1 + bv1, 0.0)
                            rows_bf[e, pl.ds(0, 32)] = plsc.pack(
                                m0, m1, format=plsc.PackFormat.INTERLEAVED)
                        return 0
                    lax.fori_loop(0, LW // 16, grp_body, 0)
                    pltpu.sync_copy(rows_bf, accum.at[dstb.at[jw]], add=True)
                    return 0
                lax.fori_loop(0, CH, win_body, 0)
                return 0
            lax.fori_loop(0, nch, chunk_body, 0)
            plsc.subcore_barrier()
            pltpu.sync_copy(accum.at[pl.ds(s * RPT, RPT)],
                            out_hbm.at[blk, pl.ds(s * RPT, RPT)])

    return k(h_flat, src2d, dst2d, attr1d, w128, b128)


def _mlp0_body(x_ref, a_ref, w1_ref, b1_ref, w2_ref, b2_ref, eps_ref, o_ref):
    aggr = a_ref[0] + a_ref[1]
    h = (1.0 + eps_ref[0]) * x_ref[...] + aggr[:, :6]
    z = jnp.dot(h, w1_ref[...], preferred_element_type=jnp.float32) + b1_ref[...]
    z = jnp.maximum(z, 0.0)
    z = jnp.dot(z, w2_ref[...], preferred_element_type=jnp.float32) + b2_ref[...]
    z = jnp.maximum(z, 0.0)
    for b in range(4):
        o_ref[b] = z[:, b * 32:(b + 1) * 32]


def _mlp0_block(x, aggr_parts, W1, B1, W2, B2, eps):
    grid = (N // ROWS,)
    return pl.pallas_call(
        _mlp0_body,
        grid=grid,
        in_specs=[
            pl.BlockSpec((ROWS, 6), lambda i: (i, 0)),
            pl.BlockSpec((2, ROWS, 16), lambda i: (0, i, 0)),
            pl.BlockSpec(W1.shape, lambda i: (0, 0)),
            pl.BlockSpec((1, 2 * H), lambda i: (0, 0)),
            pl.BlockSpec(W2.shape, lambda i: (0, 0)),
            pl.BlockSpec((1, H), lambda i: (0, 0)),
            pl.BlockSpec(memory_space=pltpu.SMEM),
        ],
        out_specs=pl.BlockSpec((4, ROWS, 32), lambda i: (0, i, 0)),
        out_shape=jax.ShapeDtypeStruct((4, N, 32), jnp.float32),
    )(x, aggr_parts, W1, B1[None, :], W2, B2[None, :], eps.reshape(1))


def _mlp1_pool_body(h_ref, a_ref, w1_ref, b1_ref, w2_ref, b2_ref, eps_ref,
                    batch_ref, sums_ref, cnt_ref):
    i = pl.program_id(0)
    hin = jnp.concatenate([h_ref[b] for b in range(4)], axis=1)
    aggr = jnp.concatenate([a_ref[b] for b in range(4)], axis=1)
    h = (1.0 + eps_ref[0]) * hin + aggr
    z = jnp.dot(h, w1_ref[...], preferred_element_type=jnp.float32) + b1_ref[...]
    z = jnp.maximum(z, 0.0)
    z = jnp.dot(z, w2_ref[...], preferred_element_type=jnp.float32) + b2_ref[...]
    z = jnp.maximum(z, 0.0)
    gids = jax.lax.broadcasted_iota(jnp.int32, (ROWS, G), 1)
    oh = (batch_ref[...] == gids).astype(jnp.float32)
    part = jnp.dot(oh.T, z, preferred_element_type=jnp.float32)
    pcnt = jnp.sum(oh, axis=0)

    @pl.when(i == 0)
    def _init():
        sums_ref[...] = jnp.zeros_like(sums_ref)
        cnt_ref[...] = jnp.zeros_like(cnt_ref)

    sums_ref[...] += part
    cnt_ref[...] += pcnt[None, :]


def _mlp1_pool_block(h_tables, aggr_tables, W1, B1, W2, B2, eps, batch2d):
    grid = (N // ROWS,)
    return pl.pallas_call(
        _mlp1_pool_body,
        grid=grid,
        in_specs=[
            pl.BlockSpec((4, ROWS, 32), lambda i: (0, i, 0)),
            pl.BlockSpec((4, ROWS, 32), lambda i: (0, i, 0)),
            pl.BlockSpec(W1.shape, lambda i: (0, 0)),
            pl.BlockSpec((1, 2 * H), lambda i: (0, 0)),
            pl.BlockSpec(W2.shape, lambda i: (0, 0)),
            pl.BlockSpec((1, H), lambda i: (0, 0)),
            pl.BlockSpec(memory_space=pltpu.SMEM),
            pl.BlockSpec((ROWS, 1), lambda i: (i, 0)),
        ],
        out_specs=[
            pl.BlockSpec((G, H), lambda i: (0, 0)),
            pl.BlockSpec((1, G), lambda i: (0, 0)),
        ],
        out_shape=[
            jax.ShapeDtypeStruct((G, H), jnp.float32),
            jax.ShapeDtypeStruct((1, G), jnp.float32),
        ],
    )(h_tables, aggr_tables, W1, B1[None, :], W2, B2[None, :], eps.reshape(1),
      batch2d)


def kernel(x, edge_index, edge_attr, batch, paper_count, params):
    src, dst = edge_index[0], edge_index[1]
    p0, p1 = params['conv0'], params['conv1']

    W1a, B1a = _fold_bn(p0['w1'], p0['b1'], p0['bn1_g'], p0['bn1_b'],
                        p0['bn1_m'], p0['bn1_v'])
    W2a, B2a = _fold_bn(p0['w2'], p0['b2'], p0['bn2_g'], p0['bn2_b'],
                        p0['bn2_m'], p0['bn2_v'])
    W1b, B1b = _fold_bn(p1['w1'], p1['b1'], p1['bn1_g'], p1['bn1_b'],
                        p1['bn1_m'], p1['bn1_v'])
    W2b, B2b = _fold_bn(p1['w2'], p1['b2'], p1['bn2_g'], p1['bn2_b'],
                        p1['bn2_m'], p1['bn2_v'])

    P = EPAD - E
    src_p = jnp.concatenate([src, jnp.zeros((P,), jnp.int32)]
                            ).reshape(NWIN, LW)
    dst_p = jnp.concatenate(
        [dst, (N + jnp.arange(P, dtype=jnp.int32) % NPADROW)]
    ).reshape(NWIN, LW)
    attr_p = jnp.concatenate([edge_attr[:, 0], jnp.zeros((P,), jnp.float32)])

    x16 = jnp.pad(x, ((0, 0), (0, 10)))
    w16 = jnp.pad(p0['el_w'][0], (0, 10))
    b16 = jnp.pad(p0['el_b'], (0, 10))
    aggr0_parts = _sc_aggr_layer0(x16, src_p, dst_p, attr_p, w16, b16)
    h_tables = _mlp0_block(x, aggr0_parts, W1a, B1a, W2a, B2a, p0['eps'])

    h_flat = h_tables.reshape(4 * N, 32)
    aggr1_bf = _sc_aggr_layer1(h_flat, src_p, dst_p, attr_p,
                               p1['el_w'][0], p1['el_b'])
    inv = [2 * k if k < 16 else 2 * (k - 16) + 1 for k in range(32)]
    aggr1_tables = aggr1_bf[:, :, jnp.array(inv)].astype(jnp.float32)
    sums, cnt = _mlp1_pool_block(h_tables, aggr1_tables, W1b, B1b, W2b, B2b,
                                 p1['eps'], batch[:, None])

    pooled = sums / jnp.clip(cnt[0], 1.0)[:, None]
    logits = pooled @ params['lin_w'] + params['lin_b']
    return jax.nn.log_softmax(logits, axis=-1)


# speedup vs baseline: 1.3563x; 1.3563x over previous
import functools

import jax
import jax.numpy as jnp
from jax import lax
from jax.experimental import pallas as pl
from jax.experimental.pallas import tpu as pltpu
from jax.experimental.pallas import tpu_sc as plsc

N = 50000
E = 800000
G = 512
H = 128
ROWS = 400

LW = 128
EPAD = 819200
NWIN = EPAD // LW
NACC = 51200
RPT = NACC // 16
CH = 40
NPADROW = NACC - N


def _fold_bn(w, b, g, bb, m, v):
    s = g / jnp.sqrt(v + 1e-5)
    W = w * s[None, :]
    B = b * s + bb - m * s
    return W, B


def _zero_vmem(ref, rows, cols):
    def body(r, _):
        for c16 in range(cols // 16):
            ref[r, pl.ds(16 * c16, 16)] = jnp.zeros((16,), jnp.float32)
        return 0
    lax.fori_loop(0, rows, body, 0)


def _zero_accum_slice(zbuf, accum, s):
    def body(k, _):
        pltpu.sync_copy(zbuf, accum.at[pl.ds(s * RPT + k * LW, LW)])
        return 0
    lax.fori_loop(0, RPT // LW, body, 0)


def _sc_aggr_layer0(x16, src2d, dst2d, attr1d, w16, b16):
    mesh = plsc.VectorSubcoreMesh(core_axis_name="c", subcore_axis_name="s")
    wpt = NWIN // 32
    nch = wpt // CH

    @functools.partial(
        pl.kernel, mesh=mesh,
        out_type=jax.ShapeDtypeStruct((2, NACC, 16), jnp.float32),
        compiler_params=pltpu.CompilerParams(use_tc_tiling_on_sc=False),
        scratch_types=[
            pltpu.VMEM((CH, LW), jnp.int32),
            pltpu.VMEM((CH, LW), jnp.int32),
            pltpu.VMEM((CH * LW,), jnp.float32),
            pltpu.VMEM((LW, 16), jnp.float32),
            pltpu.VMEM((LW, 16), jnp.float32),
            pltpu.VMEM((LW, 16), jnp.float32),
            pltpu.VMEM((LW,), jnp.int32),
            pltpu.VMEM((16,), jnp.float32),
            pltpu.VMEM((16,), jnp.float32),
            pltpu.VMEM_SHARED((NACC, 16), jnp.float32),
            pltpu.SemaphoreType.DMA,
            pltpu.SemaphoreType.DMA,
            pltpu.SemaphoreType.DMA,
        ],
    )
    def k(x_hbm, src_hbm, dst_hbm, attr_hbm, w_hbm, b_hbm, out_hbm,
          srcb, dstb, attrb, rows, sbuf0, sbuf1, zidx, wbuf, bbuf,
          accum, semg, sems0, sems1):
        c = lax.axis_index("c")
        s = lax.axis_index("s")
        wid = s * 2 + c
        sbufs = (sbuf0, sbuf1)
        sems = (sems0, sems1)
        pltpu.sync_copy(w_hbm, wbuf)
        pltpu.sync_copy(b_hbm, bbuf)
        wv = wbuf[pl.ds(0, 16)]
        bv = bbuf[pl.ds(0, 16)]
        _zero_vmem(sbuf0, LW, 16)
        _zero_vmem(sbuf1, LW, 16)
        for i8 in range(LW // 16):
            zidx[pl.ds(16 * i8, 16)] = jnp.zeros((16,), jnp.int32)
        _zero_accum_slice(sbuf0, accum, s)
        plsc.subcore_barrier()
        for b in range(2):
            pltpu.make_async_copy(sbufs[b], accum.at[zidx],
                                  sems[b]).start(add=True)

        def chunk_body(jc, _):
            row0 = wid * wpt + jc * CH
            pltpu.sync_copy(src_hbm.at[pl.ds(row0, CH)], srcb)
            pltpu.sync_copy(dst_hbm.at[pl.ds(row0, CH)], dstb)
            pltpu.sync_copy(attr_hbm.at[pl.ds(row0 * LW, CH * LW)], attrb)

            def win2_body(jj, _):
                for b in range(2):
                    jw = jj * 2 + b
                    sb = sbufs[b]
                    pltpu.async_copy(x_hbm.at[srcb.at[jw]], rows, semg).wait()
                    pltpu.make_async_copy(sb, accum.at[zidx], sems[b]).wait()

                    def grp_body(g, _):
                        av = attrb[pl.ds(jw * LW + g * 16, 16)]
                        for i in range(16):
                            e = g * 16 + i
                            a = av[i]
                            v = rows[e, pl.ds(0, 16)]
                            sb[e, pl.ds(0, 16)] = jnp.maximum(v + a * wv + bv,
                                                              0.0)
                        return 0
                    lax.fori_loop(0, LW // 16, grp_body, 0)
                    pltpu.make_async_copy(
                        sb, accum.at[dstb.at[jw]], sems[b]).start(add=True)
                return 0
            lax.fori_loop(0, CH // 2, win2_body, 0)
            return 0
        lax.fori_loop(0, nch, chunk_body, 0)
        for b in range(2):
            pltpu.make_async_copy(sbufs[b], accum.at[zidx], sems[b]).wait()
        plsc.subcore_barrier()
        pltpu.sync_copy(accum.at[pl.ds(s * RPT, RPT)],
                        out_hbm.at[c, pl.ds(s * RPT, RPT)])

    return k(x16, src2d, dst2d, attr1d, w16, b16)


def _sc_aggr_layer1(h_flat, src2d, dst2d, attr1d, w128, b128):
    mesh = plsc.VectorSubcoreMesh(core_axis_name="c", subcore_axis_name="s")
    wpt = NWIN // 16
    nch = wpt // CH

    @functools.partial(
        pl.kernel, mesh=mesh,
        out_type=jax.ShapeDtypeStruct((4, NACC, 32), jnp.float32),
        compiler_params=pltpu.CompilerParams(use_tc_tiling_on_sc=False),
        scratch_types=[
            pltpu.VMEM((CH, LW), jnp.int32),
            pltpu.VMEM((CH, LW), jnp.int32),
            pltpu.VMEM((CH * LW,), jnp.float32),
            pltpu.VMEM((LW, 32), jnp.float32),
            pltpu.VMEM((LW, 32), jnp.float32),
            pltpu.VMEM((LW, 32), jnp.float32),
            pltpu.VMEM((LW,), jnp.int32),
            pltpu.VMEM((32,), jnp.float32),
            pltpu.VMEM((32,), jnp.float32),
            pltpu.VMEM_SHARED((NACC, 32), jnp.float32),
            pltpu.SemaphoreType.DMA,
            pltpu.SemaphoreType.DMA,
            pltpu.SemaphoreType.DMA,
        ],
    )
    def k(h_hbm, src_hbm, dst_hbm, attr_hbm, w_hbm, b_hbm, out_hbm,
          srcb, dstb, attrb, rows, sbuf0, sbuf1, zidx, wbuf, bbuf,
          accum, semg, sems0, sems1):
        c = lax.axis_index("c")
        s = lax.axis_index("s")
        sbufs = (sbuf0, sbuf1)
        sems = (sems0, sems1)
        for i8 in range(LW // 16):
            zidx[pl.ds(16 * i8, 16)] = jnp.zeros((16,), jnp.int32)
        for bi in range(2):
            blk = c * 2 + bi
            pltpu.sync_copy(w_hbm.at[pl.ds(blk * 32, 32)], wbuf)
            pltpu.sync_copy(b_hbm.at[pl.ds(blk * 32, 32)], bbuf)
            wv0 = wbuf[pl.ds(0, 16)]
            wv1 = wbuf[pl.ds(16, 16)]
            bv0 = bbuf[pl.ds(0, 16)]
            bv1 = bbuf[pl.ds(16, 16)]
            _zero_vmem(sbuf0, LW, 32)
            _zero_vmem(sbuf1, LW, 32)
            _zero_accum_slice(sbuf0, accum, s)
            plsc.subcore_barrier()
            for b in range(2):
                pltpu.make_async_copy(sbufs[b], accum.at[zidx],
                                      sems[b]).start(add=True)
            boff = blk * N

            def chunk_body(jc, _):
                row0 = s * wpt + jc * CH
                pltpu.sync_copy(src_hbm.at[pl.ds(row0, CH)], srcb)
                pltpu.sync_copy(dst_hbm.at[pl.ds(row0, CH)], dstb)
                pltpu.sync_copy(attr_hbm.at[pl.ds(row0 * LW, CH * LW)], attrb)

                def adj_body(jw, _):
                    for i8 in range(LW // 16):
                        sl = pl.ds(16 * i8, 16)
                        srcb[jw, sl] = srcb[jw, sl] + boff
                    return 0
                lax.fori_loop(0, CH, adj_body, 0)

                def win2_body(jj, _):
                    for b in range(2):
                        jw = jj * 2 + b
                        sb = sbufs[b]
                        pltpu.async_copy(h_hbm.at[srcb.at[jw]], rows,
                                         semg).wait()
                        pltpu.make_async_copy(sb, accum.at[zidx],
                                              sems[b]).wait()

                        def grp_body(g, _):
                            av = attrb[pl.ds(jw * LW + g * 16, 16)]
                            for i in range(16):
                                e = g * 16 + i
                                a = av[i]
                                v0 = rows[e, pl.ds(0, 16)]
                                sb[e, pl.ds(0, 16)] = jnp.maximum(
                                    v0 + a * wv0 + bv0, 0.0)
                                v1 = rows[e, pl.ds(16, 16)]
                                sb[e, pl.ds(16, 16)] = jnp.maximum(
                                    v1 + a * wv1 + bv1, 0.0)
                            return 0
                        lax.fori_loop(0, LW // 16, grp_body, 0)
                        pltpu.make_async_copy(
                            sb, accum.at[dstb.at[jw]], sems[b]).start(add=True)
                    return 0
                lax.fori_loop(0, CH // 2, win2_body, 0)
                return 0
            lax.fori_loop(0, nch, chunk_body, 0)
            for b in range(2):
                pltpu.make_async_copy(sbufs[b], accum.at[zidx],
                                      sems[b]).wait()
            plsc.subcore_barrier()
            pltpu.sync_copy(accum.at[pl.ds(s * RPT, RPT)],
                            out_hbm.at[blk, pl.ds(s * RPT, RPT)])

    return k(h_flat, src2d, dst2d, attr1d, w128, b128)


def _mlp0_body(x_ref, a_ref, w1_ref, b1_ref, w2_ref, b2_ref, eps_ref, o_ref):
    aggr = a_ref[0] + a_ref[1]
    h = (1.0 + eps_ref[0]) * x_ref[...] + aggr[:, :6]
    z = jnp.dot(h, w1_ref[...], preferred_element_type=jnp.float32) + b1_ref[...]
    z = jnp.maximum(z, 0.0)
    z = jnp.dot(z, w2_ref[...], preferred_element_type=jnp.float32) + b2_ref[...]
    z = jnp.maximum(z, 0.0)
    for b in range(4):
        o_ref[b] = z[:, b * 32:(b + 1) * 32]


def _mlp0_block(x, aggr_parts, W1, B1, W2, B2, eps):
    grid = (N // ROWS,)
    return pl.pallas_call(
        _mlp0_body,
        grid=grid,
        in_specs=[
            pl.BlockSpec((ROWS, 6), lambda i: (i, 0)),
            pl.BlockSpec((2, ROWS, 16), lambda i: (0, i, 0)),
            pl.BlockSpec(W1.shape, lambda i: (0, 0)),
            pl.BlockSpec((1, 2 * H), lambda i: (0, 0)),
            pl.BlockSpec(W2.shape, lambda i: (0, 0)),
            pl.BlockSpec((1, H), lambda i: (0, 0)),
            pl.BlockSpec(memory_space=pltpu.SMEM),
        ],
        out_specs=pl.BlockSpec((4, ROWS, 32), lambda i: (0, i, 0)),
        out_shape=jax.ShapeDtypeStruct((4, N, 32), jnp.float32),
    )(x, aggr_parts, W1, B1[None, :], W2, B2[None, :], eps.reshape(1))


def _mlp1_pool_body(h_ref, a_ref, w1_ref, b1_ref, w2_ref, b2_ref, eps_ref,
                    batch_ref, sums_ref, cnt_ref):
    i = pl.program_id(0)
    hin = jnp.concatenate([h_ref[b] for b in range(4)], axis=1)
    aggr = jnp.concatenate([a_ref[b] for b in range(4)], axis=1)
    h = (1.0 + eps_ref[0]) * hin + aggr
    z = jnp.dot(h, w1_ref[...], preferred_element_type=jnp.float32) + b1_ref[...]
    z = jnp.maximum(z, 0.0)
    z = jnp.dot(z, w2_ref[...], preferred_element_type=jnp.float32) + b2_ref[...]
    z = jnp.maximum(z, 0.0)
    gids = jax.lax.broadcasted_iota(jnp.int32, (ROWS, G), 1)
    oh = (batch_ref[...] == gids).astype(jnp.float32)
    part = jnp.dot(oh.T, z, preferred_element_type=jnp.float32)
    pcnt = jnp.sum(oh, axis=0)

    @pl.when(i == 0)
    def _init():
        sums_ref[...] = jnp.zeros_like(sums_ref)
        cnt_ref[...] = jnp.zeros_like(cnt_ref)

    sums_ref[...] += part
    cnt_ref[...] += pcnt[None, :]


def _mlp1_pool_block(h_tables, aggr_tables, W1, B1, W2, B2, eps, batch2d):
    grid = (N // ROWS,)
    return pl.pallas_call(
        _mlp1_pool_body,
        grid=grid,
        in_specs=[
            pl.BlockSpec((4, ROWS, 32), lambda i: (0, i, 0)),
            pl.BlockSpec((4, ROWS, 32), lambda i: (0, i, 0)),
            pl.BlockSpec(W1.shape, lambda i: (0, 0)),
            pl.BlockSpec((1, 2 * H), lambda i: (0, 0)),
            pl.BlockSpec(W2.shape, lambda i: (0, 0)),
            pl.BlockSpec((1, H), lambda i: (0, 0)),
            pl.BlockSpec(memory_space=pltpu.SMEM),
            pl.BlockSpec((ROWS, 1), lambda i: (i, 0)),
        ],
        out_specs=[
            pl.BlockSpec((G, H), lambda i: (0, 0)),
            pl.BlockSpec((1, G), lambda i: (0, 0)),
        ],
        out_shape=[
            jax.ShapeDtypeStruct((G, H), jnp.float32),
            jax.ShapeDtypeStruct((1, G), jnp.float32),
        ],
    )(h_tables, aggr_tables, W1, B1[None, :], W2, B2[None, :], eps.reshape(1),
      batch2d)


def kernel(x, edge_index, edge_attr, batch, paper_count, params):
    src, dst = edge_index[0], edge_index[1]
    p0, p1 = params['conv0'], params['conv1']

    W1a, B1a = _fold_bn(p0['w1'], p0['b1'], p0['bn1_g'], p0['bn1_b'],
                        p0['bn1_m'], p0['bn1_v'])
    W2a, B2a = _fold_bn(p0['w2'], p0['b2'], p0['bn2_g'], p0['bn2_b'],
                        p0['bn2_m'], p0['bn2_v'])
    W1b, B1b = _fold_bn(p1['w1'], p1['b1'], p1['bn1_g'], p1['bn1_b'],
                        p1['bn1_m'], p1['bn1_v'])
    W2b, B2b = _fold_bn(p1['w2'], p1['b2'], p1['bn2_g'], p1['bn2_b'],
                        p1['bn2_m'], p1['bn2_v'])

    P = EPAD - E
    src_p = jnp.concatenate([src, jnp.zeros((P,), jnp.int32)]
                            ).reshape(NWIN, LW)
    dst_p = jnp.concatenate(
        [dst, (N + jnp.arange(P, dtype=jnp.int32) % NPADROW)]
    ).reshape(NWIN, LW)
    attr_p = jnp.concatenate([edge_attr[:, 0], jnp.zeros((P,), jnp.float32)])

    x16 = jnp.pad(x, ((0, 0), (0, 10)))
    w16 = jnp.pad(p0['el_w'][0], (0, 10))
    b16 = jnp.pad(p0['el_b'], (0, 10))
    aggr0_parts = _sc_aggr_layer0(x16, src_p, dst_p, attr_p, w16, b16)
    h_tables = _mlp0_block(x, aggr0_parts, W1a, B1a, W2a, B2a, p0['eps'])

    h_flat = h_tables.reshape(4 * N, 32)
    aggr1_tables = _sc_aggr_layer1(h_flat, src_p, dst_p, attr_p,
                                   p1['el_w'][0], p1['el_b'])
    sums, cnt = _mlp1_pool_block(h_tables, aggr1_tables, W1b, B1b, W2b, B2b,
                                 p1['eps'], batch[:, None])

    pooled = sums / jnp.clip(cnt[0], 1.0)[:, None]
    logits = pooled @ params['lin_w'] + params['lin_b']
    return jax.nn.log_softmax(logits, axis=-1)
